# baseline (speedup 1.0000x reference)
import functools

import jax
import jax.numpy as jnp
from jax import lax
from jax.experimental import pallas as pl
from jax.experimental.pallas import tpu as pltpu
from jax.experimental.pallas import tpu_sc as plsc

_B, _N, _F, _BT, _V, _D = 16, 2048, 8, 192, 1000000, 32
_NUM_GSPS = 360
_R = _BT // _B
_C = 5 * _F + 2 + _D

_NW = 32
_BPW = (_B * _N) // _NW


_WROWS = _BPW * _D // 128


def _sc_gather_body(idx_hbm, table_hbm, out_hbm, idx_v, rows_v, sem):
    wid = lax.axis_index("s") * 2 + lax.axis_index("c")
    base = wid * _BPW
    pltpu.sync_copy(idx_hbm.at[pl.ds(base, _BPW)], idx_v)

    def fire_chunk(k, carry):
        chunk = idx_v[pl.ds(k * 16, 16)] + _NUM_GSPS
        for j in range(16):
            dst = rows_v.at[k * 4 + j // 4, pl.ds((j % 4) * _D, _D)]
            pltpu.make_async_copy(table_hbm.at[chunk[j]], dst, sem).start()
        return carry

    lax.fori_loop(0, _BPW // 16, fire_chunk, 0)
    pltpu.make_async_copy(out_hbm.at[pl.ds(wid * _WROWS, _WROWS)],
                          rows_v, sem).wait()
    pltpu.sync_copy(rows_v, out_hbm.at[pl.ds(wid * _WROWS, _WROWS)])


@functools.lru_cache(maxsize=1)
def _gather_call():
    return functools.partial(
        pl.kernel,
        out_type=jax.ShapeDtypeStruct((_B * _N * _D // 128, 128), jnp.float32),
        mesh=plsc.VectorSubcoreMesh(core_axis_name="c", subcore_axis_name="s"),
        scratch_types=[
            pltpu.VMEM((_BPW,), jnp.int32),
            pltpu.VMEM((_WROWS, 128), jnp.float32),
            pltpu.SemaphoreType.DMA,
        ],
    )(_sc_gather_body)


def _asm_body(y_ref, x_ref, emb_ref, tf_ref, tf0_ref, az_ref, el_ref,
              out_ref, base_ref):
    zeros8 = jnp.zeros((_N, _F), jnp.float32)
    zeros2 = jnp.zeros((_N, 2), jnp.float32)
    tf0_b = jnp.broadcast_to(tf0_ref[0], (_N, _F))
    base_ref[...] = jnp.concatenate(
        [zeros8, y_ref[0], x_ref[0], zeros8, tf0_b, zeros2, emb_ref[0],
         jnp.zeros((_N, 128 - _C), jnp.float32)],
        axis=-1)
    tmat = jnp.concatenate(
        [jnp.zeros((_R, 3 * _F), jnp.float32),
         tf_ref[0],
         jnp.zeros((_R, _F), jnp.float32),
         az_ref[0, 0][:, None],
         el_ref[0, 0][:, None],
         jnp.zeros((_R, _D + 128 - _C), jnp.float32)],
        axis=-1)
    for t in range(_R):
        out_ref[t] = base_ref[...] + tmat[t][None, :]


@functools.partial(jax.jit, static_argnames=())
def _assemble(y, x, emb3, tf3, tf0_3, az3, el3):
    return pl.pallas_call(
        _asm_body,
        grid=(_B,),
        in_specs=[
            pl.BlockSpec((1, _N, _F), lambda b: (b, 0, 0)),
            pl.BlockSpec((1, _N, _F), lambda b: (b, 0, 0)),
            pl.BlockSpec((1, _N, _D), lambda b: (b, 0, 0)),
            pl.BlockSpec((1, _R, _F), lambda b: (b, 0, 0)),
            pl.BlockSpec((1, 1, _F), lambda b: (b, 0, 0)),
            pl.BlockSpec((1, 1, _R), lambda b: (b, 0, 0)),
            pl.BlockSpec((1, 1, _R), lambda b: (b, 0, 0)),
        ],
        out_specs=pl.BlockSpec((_R, _N, 128), lambda b: (b, 0, 0)),
        out_shape=jax.ShapeDtypeStruct((_BT, _N, 128), jnp.float32),
        scratch_shapes=[pltpu.VMEM((_N, 128), jnp.float32)],
    )(y, x, emb3, tf3, tf0_3, az3, el3)


def kernel(pv_y_osgb_fourier, pv_x_osgb_fourier, pv_system_row_number,
           pv_x_osgb, pv_time_utc_fourier, pv_time_utc_fourier_t0,
           hrvsatellite_solar_azimuth, hrvsatellite_solar_elevation,
           emb_table):
    del pv_x_osgb
    idx_flat = pv_system_row_number.reshape(_B * _N)
    emb3 = _gather_call()(idx_flat, emb_table).reshape(_B, _N, _D)
    tf3 = pv_time_utc_fourier.reshape(_B, _R, _F)
    tf0_3 = pv_time_utc_fourier_t0.reshape(_B, 1, _F)
    az3 = hrvsatellite_solar_azimuth.reshape(_B, 1, _R)
    el3 = hrvsatellite_solar_elevation.reshape(_B, 1, _R)
    full = _assemble(pv_y_osgb_fourier, pv_x_osgb_fourier, emb3,
                     tf3, tf0_3, az3, el3)
    return full[:, :, :_C]


# speedup vs baseline: 1.3148x; 1.3148x over previous
import functools

import jax
import jax.numpy as jnp
from jax import lax
from jax.experimental import pallas as pl
from jax.experimental.pallas import tpu as pltpu
from jax.experimental.pallas import tpu_sc as plsc

_B, _N, _F, _BT, _V, _D = 16, 2048, 8, 192, 1000000, 32
_NUM_GSPS = 360
_R = _BT // _B
_C = 5 * _F + 2 + _D

_NW = 32
_BPW = (_B * _N) // _NW


_WROWS = _BPW * _D // 128


def _sc_gather_body(idx_hbm, table_hbm, out_hbm, idx_v, rows_v, sem):
    wid = lax.axis_index("s") * 2 + lax.axis_index("c")
    base = wid * _BPW
    pltpu.sync_copy(idx_hbm.at[pl.ds(base, _BPW)], idx_v)

    def fire_chunk(k, carry):
        chunk = idx_v[pl.ds(k * 16, 16)] + _NUM_GSPS
        g16 = lax.shift_right_logical(chunk, 3)
        s16 = lax.bitwise_and(chunk, 7)
        for j in range(16):
            dst = rows_v.at[k * 4 + j // 4, pl.ds((j % 4) * _D, _D)]
            pltpu.make_async_copy(table_hbm.at[g16[j], s16[j]], dst,
                                  sem).start()
        return carry

    lax.fori_loop(0, _BPW // 16, fire_chunk, 0)
    pltpu.make_async_copy(out_hbm.at[pl.ds(wid * _WROWS, _WROWS)],
                          rows_v, sem).wait()
    pltpu.sync_copy(rows_v, out_hbm.at[pl.ds(wid * _WROWS, _WROWS)])


@functools.lru_cache(maxsize=1)
def _gather_call():
    return functools.partial(
        pl.kernel,
        out_type=jax.ShapeDtypeStruct((_B * _N * _D // 128, 128), jnp.float32),
        mesh=plsc.VectorSubcoreMesh(core_axis_name="c", subcore_axis_name="s"),
        scratch_types=[
            pltpu.VMEM((_BPW,), jnp.int32),
            pltpu.VMEM((_WROWS, 128), jnp.float32),
            pltpu.SemaphoreType.DMA,
        ],
    )(_sc_gather_body)


def _asm_body(y_ref, x_ref, emb_ref, tf_ref, tf0_ref, az_ref, el_ref,
              out_ref, base_ref):
    zeros8 = jnp.zeros((_N, _F), jnp.float32)
    zeros2 = jnp.zeros((_N, 2), jnp.float32)
    tf0_b = jnp.broadcast_to(tf0_ref[0], (_N, _F))
    base_ref[...] = jnp.concatenate(
        [zeros8, y_ref[0], x_ref[0], zeros8, tf0_b, zeros2, emb_ref[0],
         jnp.zeros((_N, 128 - _C), jnp.float32)],
        axis=-1)
    tmat = jnp.concatenate(
        [jnp.zeros((_R, 3 * _F), jnp.float32),
         tf_ref[0],
         jnp.zeros((_R, _F), jnp.float32),
         az_ref[0, 0][:, None],
         el_ref[0, 0][:, None],
         jnp.zeros((_R, _D + 128 - _C), jnp.float32)],
        axis=-1)
    for t in range(_R):
        out_ref[t] = base_ref[...] + tmat[t][None, :]


@functools.partial(jax.jit, static_argnames=())
def _assemble(y, x, emb3, tf3, tf0_3, az3, el3):
    return pl.pallas_call(
        _asm_body,
        grid=(_B,),
        in_specs=[
            pl.BlockSpec((1, _N, _F), lambda b: (b, 0, 0)),
            pl.BlockSpec((1, _N, _F), lambda b: (b, 0, 0)),
            pl.BlockSpec((1, _N, _D), lambda b: (b, 0, 0)),
            pl.BlockSpec((1, _R, _F), lambda b: (b, 0, 0)),
            pl.BlockSpec((1, 1, _F), lambda b: (b, 0, 0)),
            pl.BlockSpec((1, 1, _R), lambda b: (b, 0, 0)),
            pl.BlockSpec((1, 1, _R), lambda b: (b, 0, 0)),
        ],
        out_specs=pl.BlockSpec((_R, _N, 128), lambda b: (b, 0, 0)),
        out_shape=jax.ShapeDtypeStruct((_BT, _N, 128), jnp.float32),
        scratch_shapes=[pltpu.VMEM((_N, 128), jnp.float32)],
    )(y, x, emb3, tf3, tf0_3, az3, el3)


def kernel(pv_y_osgb_fourier, pv_x_osgb_fourier, pv_system_row_number,
           pv_x_osgb, pv_time_utc_fourier, pv_time_utc_fourier_t0,
           hrvsatellite_solar_azimuth, hrvsatellite_solar_elevation,
           emb_table):
    del pv_x_osgb
    idx_flat = pv_system_row_number.reshape(_B * _N)
    table3 = emb_table.reshape(_V // 8, 8, _D)
    emb3 = _gather_call()(idx_flat, table3).reshape(_B, _N, _D)
    tf3 = pv_time_utc_fourier.reshape(_B, _R, _F)
    tf0_3 = pv_time_utc_fourier_t0.reshape(_B, 1, _F)
    az3 = hrvsatellite_solar_azimuth.reshape(_B, 1, _R)
    el3 = hrvsatellite_solar_elevation.reshape(_B, 1, _R)
    full = _assemble(pv_y_osgb_fourier, pv_x_osgb_fourier, emb3,
                     tf3, tf0_3, az3, el3)
    return full[:, :, :_C]
